# baseline (speedup 1.0000x reference)
import jax
import jax.numpy as jnp
from jax import lax
from jax.experimental import pallas as pl
from jax.experimental.pallas import tpu as pltpu
from jax.experimental.pallas import tpu_sc as plsc

N_NODES = 10000
N_EDGES = 320000

NC = 2
NS = 16
NW = NC * NS

CHUNK = 125
CPT = 80
NBUF = 2
QC = CPT // 2
ROWS_PAD = 10112
RPT = ROWS_PAD // NS


def _sc_rows(d_row):
    mesh = plsc.VectorSubcoreMesh(core_axis_name="c", subcore_axis_name="s",
                                  num_cores=NC, num_subcores=NS)
    out_type = jax.ShapeDtypeStruct((NC, ROWS_PAD, d_row), jnp.float32)
    scratch = [
        pltpu.VMEM_SHARED((ROWS_PAD, d_row), jnp.float32),
        pltpu.VMEM((QC, CHUNK), jnp.int32),
        pltpu.VMEM((QC, CHUNK), jnp.int32),
    ] + [pltpu.VMEM((CHUNK, d_row), jnp.float32) for _ in range(2)] \
      + [pltpu.SemaphoreType.DMA for _ in range(4)]

    def body(table, srcm, dstm, zrows, out, agg_sh, src_v, dst_v,
             buf0, buf1, sg0, sg1, ss0, ss1):
        c = lax.axis_index("c")
        s = lax.axis_index("s")
        wid = c * NS + s

        def g_start(j, buf, sem):
            pltpu.async_copy(table.at[src_v.at[j]], buf, sem)

        def g_wait(j, buf, sem):
            pltpu.make_async_copy(table.at[src_v.at[j]], buf, sem).wait()

        def s_start(j, buf, sem):
            pltpu.async_copy(buf, agg_sh.at[dst_v.at[j]], sem, add=True)

        def s_wait(j, buf, sem):
            pltpu.make_async_copy(buf, agg_sh.at[dst_v.at[j]], sem).wait()

        pltpu.sync_copy(zrows.at[pl.ds(s * RPT, RPT)],
                        agg_sh.at[pl.ds(s * RPT, RPT)])
        plsc.subcore_barrier()

        def half(q):
            pltpu.sync_copy(srcm.at[pl.ds(wid * CPT + q * QC, QC)], src_v)
            pltpu.sync_copy(dstm.at[pl.ds(wid * CPT + q * QC, QC)], dst_v)
            g_start(0, buf0, sg0)

            def step(g, carry):
                a = 2 * g
                g_wait(a, buf0, sg0)
                s_start(a, buf0, ss0)

                @pl.when(g > 0)
                def _():
                    s_wait(a - 1, buf1, ss1)

                g_start(a + 1, buf1, sg1)
                s_wait(a, buf0, ss0)

                @pl.when(a + 2 < QC)
                def _():
                    g_start(a + 2, buf0, sg0)

                g_wait(a + 1, buf1, sg1)
                s_start(a + 1, buf1, ss1)
                return carry

            lax.fori_loop(0, QC // 2, step, 0)
            s_wait(QC - 1, buf1, ss1)

        half(0)
        half(1)
        plsc.subcore_barrier()

        pltpu.sync_copy(agg_sh.at[pl.ds(s * RPT, RPT)],
                        out.at[c, pl.ds(s * RPT, RPT)])

    return pl.kernel(body, out_type=out_type, mesh=mesh,
                     scratch_types=scratch)


def _sc_deg():
    mesh = plsc.VectorSubcoreMesh(core_axis_name="c", subcore_axis_name="s",
                                  num_cores=NC, num_subcores=NS)
    out_type = jax.ShapeDtypeStruct((NC, ROWS_PAD, 128), jnp.float32)
    scratch = [
        pltpu.VMEM_SHARED((ROWS_PAD, 128), jnp.float32),
        pltpu.VMEM((CPT, CHUNK), jnp.int32),
        pltpu.VMEM((CHUNK, 128), jnp.float32),
    ]

    def body(dstm, zdeg_ones, out, deg_sh, dst_v, ones_v):
        c = lax.axis_index("c")
        s = lax.axis_index("s")
        wid = c * NS + s

        pltpu.sync_copy(zdeg_ones.at[pl.ds(s * RPT, RPT)],
                        deg_sh.at[pl.ds(s * RPT, RPT)])
        pltpu.sync_copy(zdeg_ones.at[pl.ds(ROWS_PAD, CHUNK)], ones_v)
        pltpu.sync_copy(dstm.at[pl.ds(wid * CPT, CPT)], dst_v)
        plsc.subcore_barrier()

        def step(j, carry):
            pltpu.sync_copy(ones_v, deg_sh.at[dst_v.at[j]], add=True)
            return carry

        lax.fori_loop(0, CPT, step, 0)
        plsc.subcore_barrier()

        pltpu.sync_copy(deg_sh.at[pl.ds(s * RPT, RPT)],
                        out.at[c, pl.ds(s * RPT, RPT)])

    return pl.kernel(body, out_type=out_type, mesh=mesh,
                     scratch_types=scratch)


_BLK = 1000


def _mid_body(x, p0a, p0b, da, db, ws0, wn0, b0, h_out):
    agg = p0a[...] + p0b[...]
    inv = 1.0 / jnp.maximum(da[...] + db[...], 1.0)
    hp = jnp.dot(x[...], ws0[...], preferred_element_type=jnp.float32,
                 precision=lax.Precision.HIGHEST)
    hn = jnp.dot(agg * inv, wn0[...], preferred_element_type=jnp.float32,
                 precision=lax.Precision.HIGHEST)
    h_out[...] = jnp.maximum(hp + hn + b0[...], 0.0)


def _final_body(h, p1a, p1b, da, db, ws1, wn1, b1, out):
    agg = p1a[...] + p1b[...]
    inv = 1.0 / jnp.maximum(da[...] + db[...], 1.0)
    sp = jnp.dot(h[...], ws1[...], preferred_element_type=jnp.float32,
                 precision=lax.Precision.HIGHEST)
    sn = jnp.dot(agg * inv, wn1[...], preferred_element_type=jnp.float32,
                 precision=lax.Precision.HIGHEST)
    out[...] = sp + sn + b1[...]


def kernel(features, edge_index, W_self0, W_neigh0, b0, W_self1, W_neigh1, b1):
    n = N_NODES
    srcm = edge_index[0].reshape(-1, CHUNK)
    dstm = edge_index[1].reshape(-1, CHUNK)
    zrows = jnp.zeros((ROWS_PAD, 128), jnp.float32)
    zdeg_ones = jnp.concatenate(
        [jnp.zeros((ROWS_PAD, 128), jnp.float32),
         jnp.ones((CHUNK, 128), jnp.float32)])

    part0 = _sc_rows(128)(features, srcm, dstm, zrows)
    pdeg = _sc_deg()(dstm, zdeg_ones)

    row_spec = pl.BlockSpec((_BLK, 128), lambda i: (i, 0))
    row64_spec = pl.BlockSpec((_BLK, 64), lambda i: (i, 0))
    deg_spec = row_spec
    w_spec = pl.BlockSpec((128, 128), lambda i: (0, 0))
    w64_spec = pl.BlockSpec((128, 64), lambda i: (0, 0))
    b_spec = pl.BlockSpec((1, 128), lambda i: (0, 0))
    b64_spec = pl.BlockSpec((1, 64), lambda i: (0, 0))

    h = pl.pallas_call(
        _mid_body,
        grid=(n // _BLK,),
        in_specs=[row_spec, row_spec, row_spec, deg_spec, deg_spec,
                  w_spec, w_spec, b_spec],
        out_specs=row_spec,
        out_shape=jax.ShapeDtypeStruct((n, 128), jnp.float32),
    )(features, part0[0], part0[1], pdeg[0], pdeg[1],
      W_self0, W_neigh0, b0.reshape(1, 128))

    part1 = _sc_rows(128)(h, srcm, dstm, zrows)

    out = pl.pallas_call(
        _final_body,
        grid=(n // _BLK,),
        in_specs=[row_spec, row_spec, row_spec, deg_spec, deg_spec,
                  w64_spec, w64_spec, b64_spec],
        out_specs=row64_spec,
        out_shape=jax.ShapeDtypeStruct((n, 64), jnp.float32),
    )(h, part1[0], part1[1], pdeg[0], pdeg[1],
      W_self1, W_neigh1, b1.reshape(1, 64))
    return out


# speedup vs baseline: 1.1586x; 1.1586x over previous
import jax
import jax.numpy as jnp
from jax import lax
from jax.experimental import pallas as pl
from jax.experimental.pallas import tpu as pltpu
from jax.experimental.pallas import tpu_sc as plsc

N_NODES = 10000
N_EDGES = 320000

NC = 2
NS = 16
NW = NC * NS

CHUNK = 125
CPT = 80
NBUF = 2
QC = CPT // 2
ROWS_PAD = 10112
RPT = ROWS_PAD // NS


def _sc_rows(d_row):
    mesh = plsc.VectorSubcoreMesh(core_axis_name="c", subcore_axis_name="s",
                                  num_cores=NC, num_subcores=NS)
    out_type = jax.ShapeDtypeStruct((NC, ROWS_PAD, d_row), jnp.float32)
    scratch = [
        pltpu.VMEM_SHARED((ROWS_PAD, d_row), jnp.float32),
        pltpu.VMEM((QC, CHUNK), jnp.int32),
        pltpu.VMEM((QC, CHUNK), jnp.int32),
    ] + [pltpu.VMEM((CHUNK, d_row), jnp.float32) for _ in range(2)] \
      + [pltpu.SemaphoreType.DMA for _ in range(2)]

    def body(table, srcm, dstm, zrows, out, agg_sh, src_v, dst_v,
             buf0, buf1, sg0, sg1):
        c = lax.axis_index("c")
        s = lax.axis_index("s")
        wid = c * NS + s

        def g_start(j, buf, sem):
            pltpu.async_copy(table.at[src_v.at[j]], buf, sem)

        def g_wait(j, buf, sem):
            pltpu.make_async_copy(table.at[src_v.at[j]], buf, sem).wait()

        pltpu.sync_copy(zrows.at[pl.ds(s * RPT, RPT)],
                        agg_sh.at[pl.ds(s * RPT, RPT)])
        plsc.subcore_barrier()

        def half(q):
            pltpu.sync_copy(srcm.at[pl.ds(wid * CPT + q * QC, QC)], src_v)
            pltpu.sync_copy(dstm.at[pl.ds(wid * CPT + q * QC, QC)], dst_v)
            g_start(0, buf0, sg0)
            g_start(1, buf1, sg1)

            def step(g, carry):
                a = 2 * g
                g_wait(a, buf0, sg0)
                pltpu.sync_copy(buf0, agg_sh.at[dst_v.at[a]], add=True)

                @pl.when(a + 2 < QC)
                def _():
                    g_start(a + 2, buf0, sg0)

                g_wait(a + 1, buf1, sg1)
                pltpu.sync_copy(buf1, agg_sh.at[dst_v.at[a + 1]], add=True)

                @pl.when(a + 3 < QC)
                def _():
                    g_start(a + 3, buf1, sg1)

                return carry

            lax.fori_loop(0, QC // 2, step, 0)

        half(0)
        half(1)
        plsc.subcore_barrier()

        pltpu.sync_copy(agg_sh.at[pl.ds(s * RPT, RPT)],
                        out.at[c, pl.ds(s * RPT, RPT)])

    return pl.kernel(body, out_type=out_type, mesh=mesh,
                     scratch_types=scratch)


def _sc_deg():
    mesh = plsc.VectorSubcoreMesh(core_axis_name="c", subcore_axis_name="s",
                                  num_cores=NC, num_subcores=NS)
    out_type = jax.ShapeDtypeStruct((NC, ROWS_PAD, 128), jnp.float32)
    scratch = [
        pltpu.VMEM_SHARED((ROWS_PAD, 128), jnp.float32),
        pltpu.VMEM((CPT, CHUNK), jnp.int32),
        pltpu.VMEM((CHUNK, 128), jnp.float32),
    ]

    def body(dstm, zdeg_ones, out, deg_sh, dst_v, ones_v):
        c = lax.axis_index("c")
        s = lax.axis_index("s")
        wid = c * NS + s

        pltpu.sync_copy(zdeg_ones.at[pl.ds(s * RPT, RPT)],
                        deg_sh.at[pl.ds(s * RPT, RPT)])
        pltpu.sync_copy(zdeg_ones.at[pl.ds(ROWS_PAD, CHUNK)], ones_v)
        pltpu.sync_copy(dstm.at[pl.ds(wid * CPT, CPT)], dst_v)
        plsc.subcore_barrier()

        def step(j, carry):
            pltpu.sync_copy(ones_v, deg_sh.at[dst_v.at[j]], add=True)
            return carry

        lax.fori_loop(0, CPT, step, 0)
        plsc.subcore_barrier()

        pltpu.sync_copy(deg_sh.at[pl.ds(s * RPT, RPT)],
                        out.at[c, pl.ds(s * RPT, RPT)])

    return pl.kernel(body, out_type=out_type, mesh=mesh,
                     scratch_types=scratch)


_BLK = 1000


def _mid_body(x, p0a, p0b, da, db, ws0, wn0, b0, h_out):
    agg = p0a[...] + p0b[...]
    inv = 1.0 / jnp.maximum(da[...] + db[...], 1.0)
    hp = jnp.dot(x[...], ws0[...], preferred_element_type=jnp.float32,
                 precision=lax.Precision.HIGHEST)
    hn = jnp.dot(agg * inv, wn0[...], preferred_element_type=jnp.float32,
                 precision=lax.Precision.HIGHEST)
    h_out[...] = jnp.maximum(hp + hn + b0[...], 0.0)


def _final_body(h, p1a, p1b, da, db, ws1, wn1, b1, out):
    agg = p1a[...] + p1b[...]
    inv = 1.0 / jnp.maximum(da[...] + db[...], 1.0)
    sp = jnp.dot(h[...], ws1[...], preferred_element_type=jnp.float32,
                 precision=lax.Precision.HIGHEST)
    sn = jnp.dot(agg * inv, wn1[...], preferred_element_type=jnp.float32,
                 precision=lax.Precision.HIGHEST)
    out[...] = sp + sn + b1[...]


def kernel(features, edge_index, W_self0, W_neigh0, b0, W_self1, W_neigh1, b1):
    n = N_NODES
    srcm = edge_index[0].reshape(-1, CHUNK)
    dstm = edge_index[1].reshape(-1, CHUNK)
    zrows = jnp.zeros((ROWS_PAD, 128), jnp.float32)
    zdeg_ones = jnp.concatenate(
        [jnp.zeros((ROWS_PAD, 128), jnp.float32),
         jnp.ones((CHUNK, 128), jnp.float32)])

    part0 = _sc_rows(128)(features, srcm, dstm, zrows)
    pdeg = _sc_deg()(dstm, zdeg_ones)

    row_spec = pl.BlockSpec((_BLK, 128), lambda i: (i, 0))
    row64_spec = pl.BlockSpec((_BLK, 64), lambda i: (i, 0))
    deg_spec = row_spec
    w_spec = pl.BlockSpec((128, 128), lambda i: (0, 0))
    w64_spec = pl.BlockSpec((128, 64), lambda i: (0, 0))
    b_spec = pl.BlockSpec((1, 128), lambda i: (0, 0))
    b64_spec = pl.BlockSpec((1, 64), lambda i: (0, 0))

    h = pl.pallas_call(
        _mid_body,
        grid=(n // _BLK,),
        in_specs=[row_spec, row_spec, row_spec, deg_spec, deg_spec,
                  w_spec, w_spec, b_spec],
        out_specs=row_spec,
        out_shape=jax.ShapeDtypeStruct((n, 128), jnp.float32),
    )(features, part0[0], part0[1], pdeg[0], pdeg[1],
      W_self0, W_neigh0, b0.reshape(1, 128))

    part1 = _sc_rows(128)(h, srcm, dstm, zrows)

    out = pl.pallas_call(
        _final_body,
        grid=(n // _BLK,),
        in_specs=[row_spec, row_spec, row_spec, deg_spec, deg_spec,
                  w64_spec, w64_spec, b64_spec],
        out_specs=row64_spec,
        out_shape=jax.ShapeDtypeStruct((n, 64), jnp.float32),
    )(h, part1[0], part1[1], pdeg[0], pdeg[1],
      W_self1, W_neigh1, b1.reshape(1, 64))
    return out


# speedup vs baseline: 1.1695x; 1.0094x over previous
import jax
import jax.numpy as jnp
from jax import lax
from jax.experimental import pallas as pl
from jax.experimental.pallas import tpu as pltpu
from jax.experimental.pallas import tpu_sc as plsc

N_NODES = 10000
N_EDGES = 320000

NC = 2
NS = 16
NW = NC * NS

CHUNK = 125
CPT = 80
NBUF = 2
QC = CPT // 2
ROWS_PAD = 10112
RPT = ROWS_PAD // NS


def _sc_rows(d_row, with_deg=False):
    mesh = plsc.VectorSubcoreMesh(core_axis_name="c", subcore_axis_name="s",
                                  num_cores=NC, num_subcores=NS)
    part = jax.ShapeDtypeStruct((NC, ROWS_PAD, d_row), jnp.float32)
    out_type = (part, part) if with_deg else part
    scratch = [
        pltpu.VMEM_SHARED((ROWS_PAD, d_row), jnp.float32),
        pltpu.VMEM((QC, CHUNK), jnp.int32),
        pltpu.VMEM((QC, CHUNK), jnp.int32),
    ] + [pltpu.VMEM((CHUNK, d_row), jnp.float32) for _ in range(2)] \
      + [pltpu.SemaphoreType.DMA for _ in range(2)]

    def body(table, srcm, dstm, zrows, *outs_scratch):
        if with_deg:
            out, out_deg = outs_scratch[0], outs_scratch[1]
            agg_sh, src_v, dst_v, buf0, buf1, sg0, sg1 = outs_scratch[2:]
        else:
            out = outs_scratch[0]
            agg_sh, src_v, dst_v, buf0, buf1, sg0, sg1 = outs_scratch[1:]
        c = lax.axis_index("c")
        s = lax.axis_index("s")
        wid = c * NS + s

        def g_start(j, buf, sem):
            pltpu.async_copy(table.at[src_v.at[j]], buf, sem)

        def g_wait(j, buf, sem):
            pltpu.make_async_copy(table.at[src_v.at[j]], buf, sem).wait()

        pltpu.sync_copy(zrows.at[pl.ds(s * RPT, RPT)],
                        agg_sh.at[pl.ds(s * RPT, RPT)])
        plsc.subcore_barrier()

        def half(q):
            pltpu.sync_copy(srcm.at[pl.ds(wid * CPT + q * QC, QC)], src_v)
            pltpu.sync_copy(dstm.at[pl.ds(wid * CPT + q * QC, QC)], dst_v)
            g_start(0, buf0, sg0)
            g_start(1, buf1, sg1)

            def step(g, carry):
                a = 2 * g
                g_wait(a, buf0, sg0)
                pltpu.sync_copy(buf0, agg_sh.at[dst_v.at[a]], add=True)

                @pl.when(a + 2 < QC)
                def _():
                    g_start(a + 2, buf0, sg0)

                g_wait(a + 1, buf1, sg1)
                pltpu.sync_copy(buf1, agg_sh.at[dst_v.at[a + 1]], add=True)

                @pl.when(a + 3 < QC)
                def _():
                    g_start(a + 3, buf1, sg1)

                return carry

            lax.fori_loop(0, QC // 2, step, 0)

        half(0)
        half(1)
        plsc.subcore_barrier()

        pltpu.sync_copy(agg_sh.at[pl.ds(s * RPT, RPT)],
                        out.at[c, pl.ds(s * RPT, RPT)])

        if with_deg:
            pltpu.sync_copy(zrows.at[pl.ds(s * RPT, RPT)],
                            agg_sh.at[pl.ds(s * RPT, RPT)])
            pltpu.sync_copy(zrows.at[pl.ds(ROWS_PAD, CHUNK)], buf0)
            plsc.subcore_barrier()

            def dhalf(q):
                pltpu.sync_copy(dstm.at[pl.ds(wid * CPT + q * QC, QC)],
                                dst_v)

                def dstep(j, carry):
                    pltpu.sync_copy(buf0, agg_sh.at[dst_v.at[j]], add=True)
                    return carry

                lax.fori_loop(0, QC, dstep, 0)

            dhalf(0)
            dhalf(1)
            plsc.subcore_barrier()
            pltpu.sync_copy(agg_sh.at[pl.ds(s * RPT, RPT)],
                            out_deg.at[c, pl.ds(s * RPT, RPT)])

    return pl.kernel(body, out_type=out_type, mesh=mesh,
                     scratch_types=scratch)


_BLK = 1000


def _mid_body(x, p0a, p0b, da, db, ws0, wn0, b0, h_out):
    agg = p0a[...] + p0b[...]
    inv = 1.0 / jnp.maximum(da[...] + db[...], 1.0)
    hp = jnp.dot(x[...], ws0[...], preferred_element_type=jnp.float32,
                 precision=lax.Precision.HIGHEST)
    hn = jnp.dot(agg * inv, wn0[...], preferred_element_type=jnp.float32,
                 precision=lax.Precision.HIGHEST)
    h_out[...] = jnp.maximum(hp + hn + b0[...], 0.0)


def _final_body(h, p1a, p1b, da, db, ws1, wn1, b1, out):
    agg = p1a[...] + p1b[...]
    inv = 1.0 / jnp.maximum(da[...] + db[...], 1.0)
    sp = jnp.dot(h[...], ws1[...], preferred_element_type=jnp.float32,
                 precision=lax.Precision.HIGHEST)
    sn = jnp.dot(agg * inv, wn1[...], preferred_element_type=jnp.float32,
                 precision=lax.Precision.HIGHEST)
    out[...] = sp + sn + b1[...]


def kernel(features, edge_index, W_self0, W_neigh0, b0, W_self1, W_neigh1, b1):
    n = N_NODES
    srcm = edge_index[0].reshape(-1, CHUNK)
    dstm = edge_index[1].reshape(-1, CHUNK)
    zrows = jnp.concatenate(
        [jnp.zeros((ROWS_PAD, 128), jnp.float32),
         jnp.ones((CHUNK, 128), jnp.float32)])

    part0, pdeg = _sc_rows(128, with_deg=True)(features, srcm, dstm, zrows)

    row_spec = pl.BlockSpec((_BLK, 128), lambda i: (i, 0))
    row64_spec = pl.BlockSpec((_BLK, 64), lambda i: (i, 0))
    deg_spec = row_spec
    w_spec = pl.BlockSpec((128, 128), lambda i: (0, 0))
    w64_spec = pl.BlockSpec((128, 64), lambda i: (0, 0))
    b_spec = pl.BlockSpec((1, 128), lambda i: (0, 0))
    b64_spec = pl.BlockSpec((1, 64), lambda i: (0, 0))

    h = pl.pallas_call(
        _mid_body,
        grid=(n // _BLK,),
        in_specs=[row_spec, row_spec, row_spec, deg_spec, deg_spec,
                  w_spec, w_spec, b_spec],
        out_specs=row_spec,
        out_shape=jax.ShapeDtypeStruct((n, 128), jnp.float32),
    )(features, part0[0], part0[1], pdeg[0], pdeg[1],
      W_self0, W_neigh0, b0.reshape(1, 128))

    part1 = _sc_rows(128)(h, srcm, dstm, zrows)

    out = pl.pallas_call(
        _final_body,
        grid=(n // _BLK,),
        in_specs=[row_spec, row_spec, row_spec, deg_spec, deg_spec,
                  w64_spec, w64_spec, b64_spec],
        out_specs=row64_spec,
        out_shape=jax.ShapeDtypeStruct((n, 64), jnp.float32),
    )(h, part1[0], part1[1], pdeg[0], pdeg[1],
      W_self1, W_neigh1, b1.reshape(1, 64))
    return out


# speedup vs baseline: 1.2151x; 1.0389x over previous
import jax
import jax.numpy as jnp
from jax import lax
from jax.experimental import pallas as pl
from jax.experimental.pallas import tpu as pltpu
from jax.experimental.pallas import tpu_sc as plsc

N_NODES = 10000
N_EDGES = 320000

NC = 2
NS = 16
NW = NC * NS

CHUNK = 125
CPT = 80
NBUF = 2
QC = CPT // 2
ROWS_PAD = 10112
RPT = ROWS_PAD // NS


def _sc_rows(d_row, with_deg=False):
    mesh = plsc.VectorSubcoreMesh(core_axis_name="c", subcore_axis_name="s",
                                  num_cores=NC, num_subcores=NS)
    part = jax.ShapeDtypeStruct((NC, ROWS_PAD, d_row), jnp.float32)
    out_type = (part, part) if with_deg else part
    scratch = [
        pltpu.VMEM_SHARED((ROWS_PAD, d_row), jnp.float32),
        pltpu.VMEM((QC, CHUNK), jnp.int32),
        pltpu.VMEM((QC, CHUNK), jnp.int32),
    ] + [pltpu.VMEM((CHUNK, d_row), jnp.float32) for _ in range(2)] \
      + [pltpu.SemaphoreType.DMA for _ in range(2)]

    def body(table, srcm, dstm, zrows, *outs_scratch):
        if with_deg:
            out, out_deg = outs_scratch[0], outs_scratch[1]
            agg_sh, src_v, dst_v, buf0, buf1, sg0, sg1 = outs_scratch[2:]
        else:
            out = outs_scratch[0]
            agg_sh, src_v, dst_v, buf0, buf1, sg0, sg1 = outs_scratch[1:]
        c = lax.axis_index("c")
        s = lax.axis_index("s")
        wid = c * NS + s

        def g_start(j, buf, sem):
            pltpu.async_copy(table.at[src_v.at[j]], buf, sem)

        def g_wait(j, buf, sem):
            pltpu.make_async_copy(table.at[src_v.at[j]], buf, sem).wait()

        pltpu.sync_copy(zrows.at[pl.ds(s * RPT, RPT)],
                        agg_sh.at[pl.ds(s * RPT, RPT)])
        plsc.subcore_barrier()

        def half(q):
            pltpu.sync_copy(srcm.at[pl.ds(wid * CPT + q * QC, QC)], src_v)
            pltpu.sync_copy(dstm.at[pl.ds(wid * CPT + q * QC, QC)], dst_v)
            g_start(0, buf0, sg0)
            g_start(1, buf1, sg1)

            def step(g, carry):
                a = 2 * g
                g_wait(a, buf0, sg0)
                pltpu.sync_copy(buf0, agg_sh.at[dst_v.at[a]], add=True)

                @pl.when(a + 2 < QC)
                def _():
                    g_start(a + 2, buf0, sg0)

                g_wait(a + 1, buf1, sg1)
                pltpu.sync_copy(buf1, agg_sh.at[dst_v.at[a + 1]], add=True)

                @pl.when(a + 3 < QC)
                def _():
                    g_start(a + 3, buf1, sg1)

                return carry

            lax.fori_loop(0, QC // 2, step, 0)

        half(0)
        half(1)
        plsc.subcore_barrier()

        pltpu.sync_copy(agg_sh.at[pl.ds(s * RPT, RPT)],
                        out.at[c, pl.ds(s * RPT, RPT)])

        if with_deg:
            pltpu.sync_copy(zrows.at[pl.ds(s * RPT, RPT)],
                            agg_sh.at[pl.ds(s * RPT, RPT)])
            pltpu.sync_copy(zrows.at[pl.ds(ROWS_PAD, CHUNK)], buf0)
            plsc.subcore_barrier()

            def dhalf(q):
                pltpu.sync_copy(dstm.at[pl.ds(wid * CPT + q * QC, QC)],
                                dst_v)

                def dstep(j, carry):
                    pltpu.sync_copy(buf0, agg_sh.at[dst_v.at[j]], add=True)
                    return carry

                lax.fori_loop(0, QC, dstep, 0)

            dhalf(0)
            dhalf(1)
            plsc.subcore_barrier()
            pltpu.sync_copy(agg_sh.at[pl.ds(s * RPT, RPT)],
                            out_deg.at[c, pl.ds(s * RPT, RPT)])

    return pl.kernel(body, out_type=out_type, mesh=mesh,
                     scratch_types=scratch)


_BLK = 1000


def _mid_body(x, p0a, p0b, da, db, ws0, wn0, b0, h_out):
    agg = p0a[...] + p0b[...]
    inv = 1.0 / jnp.maximum(da[...] + db[...], 1.0)
    hp = jnp.dot(x[...], ws0[...], preferred_element_type=jnp.float32)
    hn = jnp.dot(agg * inv, wn0[...], preferred_element_type=jnp.float32)
    h_out[...] = jnp.maximum(hp + hn + b0[...], 0.0)


def _final_body(h, p1a, p1b, da, db, ws1, wn1, b1, out):
    agg = p1a[...] + p1b[...]
    inv = 1.0 / jnp.maximum(da[...] + db[...], 1.0)
    sp = jnp.dot(h[...], ws1[...], preferred_element_type=jnp.float32)
    sn = jnp.dot(agg * inv, wn1[...], preferred_element_type=jnp.float32)
    out[...] = sp + sn + b1[...]


def kernel(features, edge_index, W_self0, W_neigh0, b0, W_self1, W_neigh1, b1):
    n = N_NODES
    srcm = edge_index[0].reshape(-1, CHUNK)
    dstm = edge_index[1].reshape(-1, CHUNK)
    zrows = jnp.concatenate(
        [jnp.zeros((ROWS_PAD, 128), jnp.float32),
         jnp.ones((CHUNK, 128), jnp.float32)])

    part0, pdeg = _sc_rows(128, with_deg=True)(features, srcm, dstm, zrows)

    row_spec = pl.BlockSpec((_BLK, 128), lambda i: (i, 0))
    row64_spec = pl.BlockSpec((_BLK, 64), lambda i: (i, 0))
    deg_spec = row_spec
    w_spec = pl.BlockSpec((128, 128), lambda i: (0, 0))
    w64_spec = pl.BlockSpec((128, 64), lambda i: (0, 0))
    b_spec = pl.BlockSpec((1, 128), lambda i: (0, 0))
    b64_spec = pl.BlockSpec((1, 64), lambda i: (0, 0))

    h = pl.pallas_call(
        _mid_body,
        grid=(n // _BLK,),
        in_specs=[row_spec, row_spec, row_spec, deg_spec, deg_spec,
                  w_spec, w_spec, b_spec],
        out_specs=row_spec,
        out_shape=jax.ShapeDtypeStruct((n, 128), jnp.float32),
    )(features, part0[0], part0[1], pdeg[0], pdeg[1],
      W_self0, W_neigh0, b0.reshape(1, 128))

    part1 = _sc_rows(128)(h, srcm, dstm, zrows)

    out = pl.pallas_call(
        _final_body,
        grid=(n // _BLK,),
        in_specs=[row_spec, row_spec, row_spec, deg_spec, deg_spec,
                  w64_spec, w64_spec, b64_spec],
        out_specs=row64_spec,
        out_shape=jax.ShapeDtypeStruct((n, 64), jnp.float32),
    )(h, part1[0], part1[1], pdeg[0], pdeg[1],
      W_self1, W_neigh1, b1.reshape(1, 64))
    return out


# speedup vs baseline: 1.2630x; 1.0394x over previous
import jax
import jax.numpy as jnp
import numpy as np
from jax import lax
from jax.experimental import pallas as pl
from jax.experimental.pallas import tpu as pltpu
from jax.experimental.pallas import tpu_sc as plsc

N_NODES = 10000
N_EDGES = 320000

NC = 2
NS = 16
NW = NC * NS

CHUNK = 125
CPT = 80
NBUF = 2
QC = CPT // 2
ROWS_PAD = 10112
RPT = ROWS_PAD // NS


def _sc_rows(d_row, with_deg=False):
    mesh = plsc.VectorSubcoreMesh(core_axis_name="c", subcore_axis_name="s",
                                  num_cores=NC, num_subcores=NS)
    part = jax.ShapeDtypeStruct((NC, ROWS_PAD, d_row), jnp.float32)
    out_type = (part, part) if with_deg else part
    scratch = [
        pltpu.VMEM_SHARED((ROWS_PAD, d_row), jnp.float32),
        pltpu.VMEM((QC, CHUNK), jnp.int32),
        pltpu.VMEM((QC, CHUNK), jnp.int32),
    ] + [pltpu.VMEM((CHUNK, d_row), jnp.float32) for _ in range(2)] \
      + [pltpu.SemaphoreType.DMA for _ in range(2)]

    def body(table, srcm, dstm, zrows, *outs_scratch):
        if with_deg:
            out, out_deg = outs_scratch[0], outs_scratch[1]
            agg_sh, src_v, dst_v, buf0, buf1, sg0, sg1 = outs_scratch[2:]
        else:
            out = outs_scratch[0]
            agg_sh, src_v, dst_v, buf0, buf1, sg0, sg1 = outs_scratch[1:]
        c = lax.axis_index("c")
        s = lax.axis_index("s")
        wid = c * NS + s

        def g_start(j, buf, sem):
            pltpu.async_copy(table.at[src_v.at[j]], buf, sem)

        def g_wait(j, buf, sem):
            pltpu.make_async_copy(table.at[src_v.at[j]], buf, sem).wait()

        pltpu.sync_copy(zrows.at[pl.ds(s * RPT, RPT)],
                        agg_sh.at[pl.ds(s * RPT, RPT)])
        plsc.subcore_barrier()

        def half(q):
            pltpu.sync_copy(srcm.at[pl.ds(wid * CPT + q * QC, QC)], src_v)
            pltpu.sync_copy(dstm.at[pl.ds(wid * CPT + q * QC, QC)], dst_v)
            g_start(0, buf0, sg0)
            g_start(1, buf1, sg1)

            def step(g, carry):
                a = 2 * g
                g_wait(a, buf0, sg0)
                pltpu.sync_copy(buf0, agg_sh.at[dst_v.at[a]], add=True)

                @pl.when(a + 2 < QC)
                def _():
                    g_start(a + 2, buf0, sg0)

                g_wait(a + 1, buf1, sg1)
                pltpu.sync_copy(buf1, agg_sh.at[dst_v.at[a + 1]], add=True)

                @pl.when(a + 3 < QC)
                def _():
                    g_start(a + 3, buf1, sg1)

                return carry

            lax.fori_loop(0, QC // 2, step, 0)

        half(0)
        half(1)
        plsc.subcore_barrier()

        pltpu.sync_copy(agg_sh.at[pl.ds(s * RPT, RPT)],
                        out.at[c, pl.ds(s * RPT, RPT)])

        if with_deg:
            pltpu.sync_copy(zrows.at[pl.ds(s * RPT, RPT)],
                            agg_sh.at[pl.ds(s * RPT, RPT)])
            pltpu.sync_copy(zrows.at[pl.ds(ROWS_PAD, CHUNK)], buf0)
            plsc.subcore_barrier()

            def dhalf(q):
                pltpu.sync_copy(dstm.at[pl.ds(wid * CPT + q * QC, QC)],
                                dst_v)

                def dstep(j, carry):
                    pltpu.sync_copy(buf0, agg_sh.at[dst_v.at[j]], add=True)
                    return carry

                lax.fori_loop(0, QC, dstep, 0)

            dhalf(0)
            dhalf(1)
            plsc.subcore_barrier()
            pltpu.sync_copy(agg_sh.at[pl.ds(s * RPT, RPT)],
                            out_deg.at[c, pl.ds(s * RPT, RPT)])

    return pl.kernel(body, out_type=out_type, mesh=mesh,
                     scratch_types=scratch)


_BLK = 1000


def _mid_body(x, p0, deg, ws0, wn0, b0, h_out):
    agg = p0[0] + p0[1]
    inv = 1.0 / jnp.maximum(deg[0] + deg[1], 1.0)
    hp = jnp.dot(x[...], ws0[...], preferred_element_type=jnp.float32)
    hn = jnp.dot(agg * inv, wn0[...], preferred_element_type=jnp.float32)
    h_out[...] = jnp.maximum(hp + hn + b0[...], 0.0)


def _final_body(h, p1, deg, ws1, wn1, b1, out):
    agg = p1[0] + p1[1]
    inv = 1.0 / jnp.maximum(deg[0] + deg[1], 1.0)
    sp = jnp.dot(h[...], ws1[...], preferred_element_type=jnp.float32)
    sn = jnp.dot(agg * inv, wn1[...], preferred_element_type=jnp.float32)
    out[...] = sp + sn + b1[...]


def kernel(features, edge_index, W_self0, W_neigh0, b0, W_self1, W_neigh1, b1):
    n = N_NODES
    srcm = edge_index[0].reshape(-1, CHUNK)
    dstm = edge_index[1].reshape(-1, CHUNK)
    zrows = jnp.asarray(np.concatenate(
        [np.zeros((ROWS_PAD, 128), np.float32),
         np.ones((CHUNK, 128), np.float32)]))

    part0, pdeg = _sc_rows(128, with_deg=True)(features, srcm, dstm, zrows)

    row_spec = pl.BlockSpec((_BLK, 128), lambda i: (i, 0))
    row64_spec = pl.BlockSpec((_BLK, 64), lambda i: (i, 0))
    part_spec = pl.BlockSpec((NC, _BLK, 128), lambda i: (0, i, 0))
    w_spec = pl.BlockSpec((128, 128), lambda i: (0, 0))
    w64_spec = pl.BlockSpec((128, 64), lambda i: (0, 0))
    b_spec = pl.BlockSpec((1, 128), lambda i: (0, 0))
    b64_spec = pl.BlockSpec((1, 64), lambda i: (0, 0))

    h = pl.pallas_call(
        _mid_body,
        grid=(n // _BLK,),
        in_specs=[row_spec, part_spec, part_spec, w_spec, w_spec, b_spec],
        out_specs=row_spec,
        out_shape=jax.ShapeDtypeStruct((n, 128), jnp.float32),
    )(features, part0, pdeg, W_self0, W_neigh0, b0.reshape(1, 128))

    part1 = _sc_rows(128)(h, srcm, dstm, zrows)

    out = pl.pallas_call(
        _final_body,
        grid=(n // _BLK,),
        in_specs=[row_spec, part_spec, part_spec, w64_spec, w64_spec,
                  b64_spec],
        out_specs=row64_spec,
        out_shape=jax.ShapeDtypeStruct((n, 64), jnp.float32),
    )(h, part1, pdeg, W_self1, W_neigh1, b1.reshape(1, 64))
    return out


# speedup vs baseline: 1.2841x; 1.0167x over previous
import jax
import jax.numpy as jnp
import numpy as np
from jax import lax
from jax.experimental import pallas as pl
from jax.experimental.pallas import tpu as pltpu
from jax.experimental.pallas import tpu_sc as plsc

N_NODES = 10000
N_EDGES = 320000

NC = 2
NS = 16
NW = NC * NS

CHUNK = 125
CPT = 80
NBUF = 2
QC = CPT // 2
ROWS_PAD = 10112
RPT = ROWS_PAD // NS


def _sc_rows(d_row, with_deg=False):
    mesh = plsc.VectorSubcoreMesh(core_axis_name="c", subcore_axis_name="s",
                                  num_cores=NC, num_subcores=NS)
    part = jax.ShapeDtypeStruct((NC, ROWS_PAD, d_row), jnp.float32)
    out_type = (part, part) if with_deg else part
    scratch = [
        pltpu.VMEM_SHARED((ROWS_PAD, d_row), jnp.float32),
        pltpu.VMEM((QC, CHUNK), jnp.int32),
        pltpu.VMEM((QC, CHUNK), jnp.int32),
    ] + [pltpu.VMEM((CHUNK, d_row), jnp.float32) for _ in range(2)] \
      + [pltpu.SemaphoreType.DMA for _ in range(2)]

    def body(table, srcm, dstm, zrows, *outs_scratch):
        if with_deg:
            out, out_deg = outs_scratch[0], outs_scratch[1]
            agg_sh, src_v, dst_v, buf0, buf1, sg0, sg1 = outs_scratch[2:]
        else:
            out = outs_scratch[0]
            agg_sh, src_v, dst_v, buf0, buf1, sg0, sg1 = outs_scratch[1:]
        c = lax.axis_index("c")
        s = lax.axis_index("s")
        wid = c * NS + s

        def g_start(j, buf, sem):
            pltpu.async_copy(table.at[src_v.at[j]], buf, sem)

        def g_wait(j, buf, sem):
            pltpu.make_async_copy(table.at[src_v.at[j]], buf, sem).wait()

        pltpu.sync_copy(zrows.at[pl.ds(s * RPT, RPT)],
                        agg_sh.at[pl.ds(s * RPT, RPT)])
        plsc.subcore_barrier()

        def half(q):
            pltpu.sync_copy(srcm.at[pl.ds(wid * CPT + q * QC, QC)], src_v)
            pltpu.sync_copy(dstm.at[pl.ds(wid * CPT + q * QC, QC)], dst_v)
            g_start(0, buf0, sg0)
            g_start(1, buf1, sg1)

            def step(g, carry):
                a = 2 * g
                g_wait(a, buf0, sg0)
                pltpu.sync_copy(buf0, agg_sh.at[dst_v.at[a]], add=True)

                @pl.when(a + 2 < QC)
                def _():
                    g_start(a + 2, buf0, sg0)

                g_wait(a + 1, buf1, sg1)
                pltpu.sync_copy(buf1, agg_sh.at[dst_v.at[a + 1]], add=True)

                @pl.when(a + 3 < QC)
                def _():
                    g_start(a + 3, buf1, sg1)

                return carry

            lax.fori_loop(0, QC // 2, step, 0)

        half(0)
        half(1)
        plsc.subcore_barrier()

        pltpu.sync_copy(agg_sh.at[pl.ds(s * RPT, RPT)],
                        out.at[c, pl.ds(s * RPT, RPT)])

        if with_deg:
            pltpu.sync_copy(zrows.at[pl.ds(s * RPT, RPT)],
                            agg_sh.at[pl.ds(s * RPT, RPT)])
            pltpu.sync_copy(zrows.at[pl.ds(ROWS_PAD, CHUNK)], buf0)
            plsc.subcore_barrier()

            def dhalf(q):
                pltpu.sync_copy(dstm.at[pl.ds(wid * CPT + q * QC, QC)],
                                dst_v)

                def dstep(j, carry):
                    pltpu.sync_copy(buf0, agg_sh.at[dst_v.at[j]], add=True)
                    return carry

                lax.fori_loop(0, QC, dstep, 0)

            dhalf(0)
            dhalf(1)
            plsc.subcore_barrier()
            pltpu.sync_copy(agg_sh.at[pl.ds(s * RPT, RPT)],
                            out_deg.at[c, pl.ds(s * RPT, RPT)])

    return pl.kernel(body, out_type=out_type, mesh=mesh,
                     scratch_types=scratch)


_BLK = 2000


def _mid_body(x, p0, deg, ws0, wn0, b0, h_out):
    agg = p0[0] + p0[1]
    inv = 1.0 / jnp.maximum(deg[0] + deg[1], 1.0)
    hp = jnp.dot(x[...], ws0[...], preferred_element_type=jnp.float32)
    hn = jnp.dot(agg * inv, wn0[...], preferred_element_type=jnp.float32)
    h_out[...] = jnp.maximum(hp + hn + b0[...], 0.0)


def _final_body(h, p1, deg, ws1, wn1, b1, out):
    agg = p1[0] + p1[1]
    inv = 1.0 / jnp.maximum(deg[0] + deg[1], 1.0)
    sp = jnp.dot(h[...], ws1[...], preferred_element_type=jnp.float32)
    sn = jnp.dot(agg * inv, wn1[...], preferred_element_type=jnp.float32)
    out[...] = sp + sn + b1[...]


def kernel(features, edge_index, W_self0, W_neigh0, b0, W_self1, W_neigh1, b1):
    n = N_NODES
    srcm = edge_index[0].reshape(-1, CHUNK)
    dstm = edge_index[1].reshape(-1, CHUNK)
    zrows = jnp.asarray(np.concatenate(
        [np.zeros((ROWS_PAD, 128), np.float32),
         np.ones((CHUNK, 128), np.float32)]))

    part0, pdeg = _sc_rows(128, with_deg=True)(features, srcm, dstm, zrows)

    row_spec = pl.BlockSpec((_BLK, 128), lambda i: (i, 0))
    row64_spec = pl.BlockSpec((_BLK, 64), lambda i: (i, 0))
    part_spec = pl.BlockSpec((NC, _BLK, 128), lambda i: (0, i, 0))
    w_spec = pl.BlockSpec((128, 128), lambda i: (0, 0))
    w64_spec = pl.BlockSpec((128, 64), lambda i: (0, 0))
    b_spec = pl.BlockSpec((1, 128), lambda i: (0, 0))
    b64_spec = pl.BlockSpec((1, 64), lambda i: (0, 0))

    h = pl.pallas_call(
        _mid_body,
        grid=(n // _BLK,),
        in_specs=[row_spec, part_spec, part_spec, w_spec, w_spec, b_spec],
        out_specs=row_spec,
        out_shape=jax.ShapeDtypeStruct((n, 128), jnp.float32),
    )(features, part0, pdeg, W_self0, W_neigh0, b0.reshape(1, 128))

    part1 = _sc_rows(128)(h, srcm, dstm, zrows)

    out = pl.pallas_call(
        _final_body,
        grid=(n // _BLK,),
        in_specs=[row_spec, part_spec, part_spec, w64_spec, w64_spec,
                  b64_spec],
        out_specs=row64_spec,
        out_shape=jax.ShapeDtypeStruct((n, 64), jnp.float32),
    )(h, part1, pdeg, W_self1, W_neigh1, b1.reshape(1, 64))
    return out


# speedup vs baseline: 1.2849x; 1.0006x over previous
import jax
import jax.numpy as jnp
import numpy as np
from jax import lax
from jax.experimental import pallas as pl
from jax.experimental.pallas import tpu as pltpu
from jax.experimental.pallas import tpu_sc as plsc

N_NODES = 10000
N_EDGES = 320000

NC = 2
NS = 16
NW = NC * NS

CHUNK = 125
CPT = 80
QC = CPT // 2
ROWS_PAD = 10112
RPT = ROWS_PAD // NS


def _sc_rows(d_row, with_deg=False):
    mesh = plsc.VectorSubcoreMesh(core_axis_name="c", subcore_axis_name="s",
                                  num_cores=NC, num_subcores=NS)
    part = jax.ShapeDtypeStruct((NC, ROWS_PAD, d_row), jnp.float32)
    out_type = (part, part) if with_deg else part
    scratch = [
        pltpu.VMEM_SHARED((ROWS_PAD, d_row), jnp.float32),
        pltpu.VMEM((QC, CHUNK), jnp.int32),
        pltpu.VMEM((QC, CHUNK), jnp.int32),
    ] + [pltpu.VMEM((CHUNK, d_row), jnp.float32) for _ in range(2)] \
      + [pltpu.SemaphoreType.DMA for _ in range(2)]

    def body(table, srcm, dstm, zrows, *outs_scratch):
        if with_deg:
            out, out_deg = outs_scratch[0], outs_scratch[1]
            agg_sh, src_v, dst_v, buf0, buf1, sg0, sg1 = outs_scratch[2:]
        else:
            out = outs_scratch[0]
            agg_sh, src_v, dst_v, buf0, buf1, sg0, sg1 = outs_scratch[1:]
        c = lax.axis_index("c")
        s = lax.axis_index("s")
        wid = c * NS + s

        def g_start(j, buf, sem):
            pltpu.async_copy(table.at[src_v.at[j]], buf, sem)

        def g_wait(j, buf, sem):
            pltpu.make_async_copy(table.at[src_v.at[j]], buf, sem).wait()

        pltpu.sync_copy(zrows.at[pl.ds(s * RPT, RPT)],
                        agg_sh.at[pl.ds(s * RPT, RPT)])
        plsc.subcore_barrier()

        def half(q):
            pltpu.sync_copy(srcm.at[pl.ds(wid * CPT + q * QC, QC)], src_v)
            pltpu.sync_copy(dstm.at[pl.ds(wid * CPT + q * QC, QC)], dst_v)
            g_start(0, buf0, sg0)
            g_start(1, buf1, sg1)

            def step(g, carry):
                a = 2 * g
                g_wait(a, buf0, sg0)
                pltpu.sync_copy(buf0, agg_sh.at[dst_v.at[a]], add=True)

                @pl.when(a + 2 < QC)
                def _():
                    g_start(a + 2, buf0, sg0)

                g_wait(a + 1, buf1, sg1)
                pltpu.sync_copy(buf1, agg_sh.at[dst_v.at[a + 1]], add=True)

                @pl.when(a + 3 < QC)
                def _():
                    g_start(a + 3, buf1, sg1)

                return carry

            lax.fori_loop(0, QC // 2, step, 0)

        half(0)
        half(1)
        plsc.subcore_barrier()

        pltpu.sync_copy(agg_sh.at[pl.ds(s * RPT, RPT)],
                        out.at[c, pl.ds(s * RPT, RPT)])

        if with_deg:
            pltpu.sync_copy(zrows.at[pl.ds(s * RPT, RPT)],
                            agg_sh.at[pl.ds(s * RPT, RPT)])
            pltpu.sync_copy(zrows.at[pl.ds(ROWS_PAD, CHUNK)], buf0)
            plsc.subcore_barrier()

            def dhalf(q):
                pltpu.sync_copy(dstm.at[pl.ds(wid * CPT + q * QC, QC)],
                                dst_v)

                def dstep(j, carry):
                    pltpu.sync_copy(buf0, agg_sh.at[dst_v.at[j]], add=True)
                    return carry

                lax.fori_loop(0, QC, dstep, 0)

            dhalf(0)
            dhalf(1)
            plsc.subcore_barrier()
            pltpu.sync_copy(agg_sh.at[pl.ds(s * RPT, RPT)],
                            out_deg.at[c, pl.ds(s * RPT, RPT)])

    return pl.kernel(body, out_type=out_type, mesh=mesh,
                     scratch_types=scratch)


_BLK = 2000


def _mid_body(x, p0, deg, ws0, wn0, b0, h_out):
    agg = p0[0] + p0[1]
    inv = 1.0 / jnp.maximum(deg[0] + deg[1], 1.0)
    hp = jnp.dot(x[...], ws0[...], preferred_element_type=jnp.float32)
    hn = jnp.dot(agg * inv, wn0[...], preferred_element_type=jnp.float32)
    h_out[...] = jnp.maximum(hp + hn + b0[...], 0.0)


def _final_body(h, p1, deg, ws1, wn1, b1, out):
    agg = p1[0] + p1[1]
    inv = 1.0 / jnp.maximum(deg[0] + deg[1], 1.0)
    sp = jnp.dot(h[...], ws1[...], preferred_element_type=jnp.float32)
    sn = jnp.dot(agg * inv, wn1[...], preferred_element_type=jnp.float32)
    out[...] = sp + sn + b1[...]


def kernel(features, edge_index, W_self0, W_neigh0, b0, W_self1, W_neigh1, b1):
    n = N_NODES
    srcm = edge_index[0].reshape(-1, CHUNK)
    dstm = edge_index[1].reshape(-1, CHUNK)
    zrows = jnp.asarray(np.concatenate(
        [np.zeros((ROWS_PAD, 128), np.float32),
         np.ones((CHUNK, 128), np.float32)]))

    part0, pdeg = _sc_rows(128, with_deg=True)(features, srcm, dstm, zrows)

    row_spec = pl.BlockSpec((_BLK, 128), lambda i: (i, 0))
    row64_spec = pl.BlockSpec((_BLK, 64), lambda i: (i, 0))
    part_spec = pl.BlockSpec((NC, _BLK, 128), lambda i: (0, i, 0))
    w_spec = pl.BlockSpec((128, 128), lambda i: (0, 0))
    w64_spec = pl.BlockSpec((128, 64), lambda i: (0, 0))
    b_spec = pl.BlockSpec((1, 128), lambda i: (0, 0))
    b64_spec = pl.BlockSpec((1, 64), lambda i: (0, 0))

    h = pl.pallas_call(
        _mid_body,
        grid=(n // _BLK,),
        in_specs=[row_spec, part_spec, part_spec, w_spec, w_spec, b_spec],
        out_specs=row_spec,
        out_shape=jax.ShapeDtypeStruct((n, 128), jnp.float32),
    )(features, part0, pdeg, W_self0, W_neigh0, b0.reshape(1, 128))

    part1 = _sc_rows(128)(h, srcm, dstm, zrows)

    out = pl.pallas_call(
        _final_body,
        grid=(n // _BLK,),
        in_specs=[row_spec, part_spec, part_spec, w64_spec, w64_spec,
                  b64_spec],
        out_specs=row64_spec,
        out_shape=jax.ShapeDtypeStruct((n, 64), jnp.float32),
    )(h, part1, pdeg, W_self1, W_neigh1, b1.reshape(1, 64))
    return out
